# baseline (speedup 1.0000x reference)
import jax
import jax.numpy as jnp
from jax import lax
from jax.experimental import pallas as pl
from jax.experimental.pallas import tpu as pltpu
from jax.experimental.pallas import tpu_sc as plsc

B = 16384
NE = 1000000
NV = 100000
ID_DIM = 64
FEAT_DIM = 32
OUT_DIM = ID_DIM + 3 * FEAT_DIM
MAX_LEN = 20
NC = 2
NS = 16
NW = NC * NS
NQ = B // NW
IC = 128
FG = 16
TQ = 32
NTC = NQ // TQ


def _feature_phase(flat_hbm, vocab, nfeat, row0, idx_ref, dring, out_hbm,
                   base, sem, sem_w):
  ngroups = nfeat // FG

  def fire_group(g, buf):
    def fire_c(c, cc):
      def fire_j(j, ccc):
        pltpu.make_async_copy(
            flat_hbm.at[pl.ds((g * FG + c) * vocab, vocab)]
            .at[idx_ref.at[pl.ds(j * IC, IC)]],
            dring.at[buf, c, pl.ds(j * IC, IC)], sem).start()
        return ccc
      lax.fori_loop(0, NQ // IC, fire_j, 0)
      return cc
    lax.fori_loop(0, FG, fire_c, 0)

  def wait_group():
    def wait_c(c, cc):
      pltpu.make_async_copy(
          flat_hbm.at[pl.ds(0, vocab)].at[idx_ref.at[pl.ds(0, IC)]],
          dring.at[0, 0, pl.ds(0, IC)], sem).wait()
      return cc
    lax.fori_loop(0, FG * (NQ // IC), wait_c, 0)

  def wait_write():
    pltpu.make_async_copy(
        dring.at[0], out_hbm.at[pl.ds(0, FG), pl.ds(base, NQ)], sem_w).wait()

  fire_group(0, 0)

  def g_body(g, c):
    buf = lax.bitwise_and(g, 1)

    @pl.when(g >= 2)
    def _():
      wait_write()

    @pl.when(g < ngroups - 1)
    def _():
      fire_group(g + 1, 1 - buf)
    wait_group()
    pltpu.make_async_copy(
        dring.at[buf],
        out_hbm.at[pl.ds(row0 + g * FG, FG), pl.ds(base, NQ)],
        sem_w).start()
    return c
  lax.fori_loop(0, ngroups, g_body, 0)

  for _ in range(2 if ngroups >= 2 else 1):
    wait_write()


def _body(qid_hbm, idflat_hbm, catflat_hbm, brandflat_hbm, tagsrm_hbm,
          bufcat_hbm, bufbrand_hbm, btflat_hbm, out_hbm,
          qid_v, cat_idx, brand_idx, tif_v, dring, tchunk, acc_t,
          sem_i, sem, sem_t, sem_w):
  wid = lax.axis_index("s") * NC + lax.axis_index("c")
  base = wid * NQ

  pltpu.sync_copy(qid_hbm.at[pl.ds(base, NQ)], qid_v)

  def fire_ent(j, c):
    pltpu.make_async_copy(
        bufcat_hbm.at[qid_v.at[pl.ds(j * IC, IC)]],
        cat_idx.at[pl.ds(j * IC, IC)], sem_i).start()
    pltpu.make_async_copy(
        bufbrand_hbm.at[qid_v.at[pl.ds(j * IC, IC)]],
        brand_idx.at[pl.ds(j * IC, IC)], sem_i).start()
    return c
  lax.fori_loop(0, NQ // IC, fire_ent, 0)

  def fire_tif(k, c):
    s = k // (NQ // IC)
    j = k - s * (NQ // IC)
    pltpu.make_async_copy(
        btflat_hbm.at[pl.ds(s * NE, NE)].at[qid_v.at[pl.ds(j * IC, IC)]],
        tif_v.at[pl.ds(s * NQ + j * IC, IC)], sem_i).start()
    return c
  lax.fori_loop(0, MAX_LEN * (NQ // IC), fire_tif, 0)

  _feature_phase(idflat_hbm, NE, ID_DIM, 0, qid_v, dring, out_hbm,
                 base, sem, sem_w)

  def wait_ent(j, c):
    pltpu.make_async_copy(
        bufcat_hbm.at[qid_v.at[pl.ds(0, IC)]],
        cat_idx.at[pl.ds(0, IC)], sem_i).wait()
    return c
  lax.fori_loop(0, (2 + MAX_LEN) * (NQ // IC), wait_ent, 0)

  _feature_phase(catflat_hbm, NV, FEAT_DIM, ID_DIM, cat_idx, dring,
                 out_hbm, base, sem, sem_w)
  _feature_phase(brandflat_hbm, NV, FEAT_DIM, ID_DIM + FEAT_DIM, brand_idx,
                 dring, out_hbm, base, sem, sem_w)

  def fire_chunk(c, buf):
    def fire_s(s, cc):
      pltpu.make_async_copy(
          tagsrm_hbm.at[tif_v.at[pl.ds(s * NQ + c * TQ, TQ)]],
          tchunk.at[buf, s], sem_t).start()
      return cc
    lax.fori_loop(0, MAX_LEN, fire_s, 0)

  def wait_chunk():
    def wait_s(s, cc):
      pltpu.make_async_copy(
          tagsrm_hbm.at[tif_v.at[pl.ds(0, TQ)]],
          tchunk.at[0, 0], sem_t).wait()
      return cc
    lax.fori_loop(0, MAX_LEN, wait_s, 0)

  fire_chunk(0, 0)
  lane16 = lax.iota(jnp.int32, 16)

  def chunk_body(c, carry):
    buf = lax.bitwise_and(c, 1)

    @pl.when(c < NTC - 1)
    def _():
      fire_chunk(c + 1, 1 - buf)
    wait_chunk()

    def red(q, cc):
      a0 = tchunk[buf, 0, q, pl.ds(0, 16)]
      a1 = tchunk[buf, 0, q, pl.ds(16, 16)]
      for s in range(1, MAX_LEN):
        a0 = a0 + tchunk[buf, s, q, pl.ds(0, 16)]
        a1 = a1 + tchunk[buf, s, q, pl.ds(16, 16)]
      qa = jnp.full((16,), c * TQ + q, jnp.int32)
      plsc.store_scatter(acc_t, [lane16, qa], a0)
      plsc.store_scatter(acc_t, [lane16 + 16, qa], a1)
      return cc
    lax.fori_loop(0, TQ, red, 0)
    return carry
  lax.fori_loop(0, NTC, chunk_body, 0)

  pltpu.sync_copy(
      acc_t, out_hbm.at[pl.ds(ID_DIM + 2 * FEAT_DIM, FEAT_DIM),
                        pl.ds(base, NQ)])


@jax.jit
def _run(query_ids, id_table, cat_table, brand_table, tags_table,
         buf_category, buf_brand, buf_tags):
  mesh = plsc.VectorSubcoreMesh(core_axis_name="c", subcore_axis_name="s")
  out_t = pl.kernel(
      _body,
      out_type=jax.ShapeDtypeStruct((OUT_DIM, B), jnp.float32),
      mesh=mesh,
      compiler_params=pltpu.CompilerParams(
          use_tc_tiling_on_sc=False, needs_layout_passes=False),
      scratch_types=[
          pltpu.VMEM((NQ,), jnp.int32),
          pltpu.VMEM((NQ,), jnp.int32),
          pltpu.VMEM((NQ,), jnp.int32),
          pltpu.VMEM((NQ * MAX_LEN,), jnp.int32),
          pltpu.VMEM((2, FG, NQ), jnp.float32),
          pltpu.VMEM((2, MAX_LEN, TQ, FEAT_DIM), jnp.float32),
          pltpu.VMEM((FEAT_DIM, NQ), jnp.float32),
          pltpu.SemaphoreType.DMA,
          pltpu.SemaphoreType.DMA,
          pltpu.SemaphoreType.DMA,
          pltpu.SemaphoreType.DMA,
      ],
  )(query_ids,
    id_table.T.reshape(-1),
    cat_table.T.reshape(-1),
    brand_table.T.reshape(-1),
    tags_table,
    buf_category, buf_brand,
    buf_tags.T.reshape(-1))
  return out_t.T


def kernel(query_ids, id_table, cat_table, brand_table, tags_table,
           buf_category, buf_brand, buf_tags):
  return _run(query_ids.astype(jnp.int32), id_table, cat_table, brand_table,
              tags_table, buf_category.astype(jnp.int32),
              buf_brand.astype(jnp.int32), buf_tags.astype(jnp.int32))


# speedup vs baseline: 3.2121x; 3.2121x over previous
import jax
import jax.numpy as jnp
from jax import lax
from jax.experimental import pallas as pl
from jax.experimental.pallas import tpu as pltpu
from jax.experimental.pallas import tpu_sc as plsc

B = 16384
NE = 1000000
ID_DIM = 64
FEAT_DIM = 32
MAX_LEN = 20
NC = 2
NS = 16
NW = NC * NS
NQ = B // NW
IC = 128
TQ = 16
ICT = 80
NTC = NQ // TQ


def _kernel_a(qid_hbm, btflat_hbm, bufcat_hbm, bufbrand_hbm,
              cattab_hbm, brandtab_hbm, tagstab_hbm,
              ocat_hbm, obrand_hbm, otags_hbm,
              qid_v, cat_idx, brand_idx, flat_pos, tif_v,
              cat_rows, brand_rows, tags_acc, tchunk,
              sem_i, sem, sem_t, sem_w):
  wid = lax.axis_index("s") * NC + lax.axis_index("c")
  base = wid * NQ

  pltpu.sync_copy(qid_hbm.at[pl.ds(base, NQ)], qid_v)

  def fire_ent(j, c):
    pltpu.make_async_copy(
        bufcat_hbm.at[qid_v.at[pl.ds(j * IC, IC)]],
        cat_idx.at[pl.ds(j * IC, IC)], sem_i).start()
    pltpu.make_async_copy(
        bufbrand_hbm.at[qid_v.at[pl.ds(j * IC, IC)]],
        brand_idx.at[pl.ds(j * IC, IC)], sem_i).start()
    return c
  lax.fori_loop(0, NQ // IC, fire_ent, 0)

  def fp_body(g, c):
    qv = qid_v[pl.ds(g * 16, 16)]
    for s in range(MAX_LEN):
      flat_pos[pl.ds(s * NQ + g * 16, 16)] = qv + s * NE
    return c
  lax.fori_loop(0, NQ // 16, fp_body, 0)

  def fire_tif(k, c):
    pltpu.make_async_copy(
        btflat_hbm.at[flat_pos.at[pl.ds(k * IC, IC)]],
        tif_v.at[pl.ds(k * IC, IC)], sem_i).start()
    return c
  lax.fori_loop(0, (NQ * MAX_LEN) // IC, fire_tif, 0)

  def wait_ent(j, c):
    pltpu.make_async_copy(
        bufcat_hbm.at[qid_v.at[pl.ds(0, IC)]],
        cat_idx.at[pl.ds(0, IC)], sem_i).wait()
    return c
  lax.fori_loop(0, 2 * (NQ // IC), wait_ent, 0)

  def fire_feat(j, buf):
    pltpu.make_async_copy(
        cattab_hbm.at[cat_idx.at[pl.ds(j * IC, IC)]],
        cat_rows.at[buf], sem).start()
    pltpu.make_async_copy(
        brandtab_hbm.at[brand_idx.at[pl.ds(j * IC, IC)]],
        brand_rows.at[buf], sem).start()

  fire_feat(0, 0)

  def feat_body(j, c):
    buf = lax.bitwise_and(j, 1)

    @pl.when(j >= 2)
    def _():
      pltpu.make_async_copy(
          cat_rows.at[0], ocat_hbm.at[pl.ds(base, IC)], sem_w).wait()
      pltpu.make_async_copy(
          brand_rows.at[0], obrand_hbm.at[pl.ds(base, IC)], sem_w).wait()

    @pl.when(j < NQ // IC - 1)
    def _():
      fire_feat(j + 1, 1 - buf)

    pltpu.make_async_copy(
        cattab_hbm.at[cat_idx.at[pl.ds(0, IC)]],
        cat_rows.at[0], sem).wait()
    pltpu.make_async_copy(
        brandtab_hbm.at[brand_idx.at[pl.ds(0, IC)]],
        brand_rows.at[0], sem).wait()
    pltpu.make_async_copy(
        cat_rows.at[buf], ocat_hbm.at[pl.ds(base + j * IC, IC)],
        sem_w).start()
    pltpu.make_async_copy(
        brand_rows.at[buf], obrand_hbm.at[pl.ds(base + j * IC, IC)],
        sem_w).start()
    return c
  lax.fori_loop(0, NQ // IC, feat_body, 0)

  def wait_tif(k, c):
    pltpu.make_async_copy(
        btflat_hbm.at[flat_pos.at[pl.ds(0, IC)]],
        tif_v.at[pl.ds(0, IC)], sem_i).wait()
    return c
  lax.fori_loop(0, (NQ * MAX_LEN) // IC, wait_tif, 0)

  NB = (TQ * MAX_LEN) // ICT

  def fire_chunk(c, buf):
    def fire_k(k, cc):
      pltpu.make_async_copy(
          tagstab_hbm.at[tif_v.at[pl.ds(c * TQ * MAX_LEN + k * ICT, ICT)]],
          tchunk.at[buf, pl.ds(k * ICT, ICT)], sem_t).start()
      return cc
    lax.fori_loop(0, NB, fire_k, 0)

  def wait_chunk():
    def wait_k(k, cc):
      pltpu.make_async_copy(
          tagstab_hbm.at[tif_v.at[pl.ds(0, ICT)]],
          tchunk.at[0, pl.ds(0, ICT)], sem_t).wait()
      return cc
    lax.fori_loop(0, NB, wait_k, 0)

  fire_chunk(0, 0)

  def chunk_body(c, carry):
    buf = lax.bitwise_and(c, 1)

    @pl.when(c < NTC - 1)
    def _():
      fire_chunk(c + 1, 1 - buf)
    wait_chunk()

    def red(q, cc):
      a0 = tchunk[buf, q * MAX_LEN, pl.ds(0, 16)]
      a1 = tchunk[buf, q * MAX_LEN, pl.ds(16, 16)]
      for s in range(1, MAX_LEN):
        a0 = a0 + tchunk[buf, q * MAX_LEN + s, pl.ds(0, 16)]
        a1 = a1 + tchunk[buf, q * MAX_LEN + s, pl.ds(16, 16)]
      tags_acc[c * TQ + q, pl.ds(0, 16)] = a0
      tags_acc[c * TQ + q, pl.ds(16, 16)] = a1
      return cc
    lax.fori_loop(0, TQ, red, 0)
    return carry
  lax.fori_loop(0, NTC, chunk_body, 0)

  for _ in range(2):
    pltpu.make_async_copy(
        cat_rows.at[0], ocat_hbm.at[pl.ds(base, IC)], sem_w).wait()
    pltpu.make_async_copy(
        brand_rows.at[0], obrand_hbm.at[pl.ds(base, IC)], sem_w).wait()
  pltpu.sync_copy(
      tags_acc, otags_hbm.at[pl.ds(base, NQ), pl.ds(0, FEAT_DIM)])


def _kernel_b(qid_hbm, idtab_hbm, oid_hbm,
              qid_v, id_out, id8, sem_q0, sem_q1, sem_w):
  wid = lax.axis_index("s") * NC + lax.axis_index("c")
  base = wid * NQ

  pltpu.sync_copy(qid_hbm.at[pl.ds(base, NQ)], qid_v)

  def fire_group(goff, par, sem_q):
    qv = qid_v[pl.ds(goff, 16)]
    for j in range(16):
      r = qv[j]
      rb = pl.multiple_of(r - lax.bitwise_and(r, 7), 8)
      pltpu.make_async_copy(
          idtab_hbm.at[pl.ds(rb, 8)], id8.at[par * 16 + j], sem_q).start()

  def drain_extract(goff, par, phase, sem_q):
    for j in range(16):
      pltpu.make_async_copy(
          idtab_hbm.at[pl.ds(0, 8)], id8.at[par * 16 + j], sem_q).wait()
    qv = qid_v[pl.ds(goff, 16)]
    for j in range(16):
      slot = par * 16 + j
      sub = lax.bitwise_and(qv[j], 7)
      for k in range(ID_DIM // 16):
        id_out[phase, par * 16 + j, pl.ds(k * 16, 16)] = (
            id8[slot, sub, pl.ds(k * 16, 16)])

  fire_group(0, 0, sem_q0)
  fire_group(16, 1, sem_q1)

  def pair_body(gg, c):
    goff = gg * 32
    phase = lax.bitwise_and(gg, 1)

    @pl.when(gg >= 2)
    def _():
      pltpu.make_async_copy(
          id_out.at[0], oid_hbm.at[pl.ds(base, 32)], sem_w).wait()

    drain_extract(goff, 0, phase, sem_q0)

    @pl.when(gg < NQ // 32 - 1)
    def _():
      fire_group(goff + 32, 0, sem_q0)
    drain_extract(goff + 16, 1, phase, sem_q1)

    @pl.when(gg < NQ // 32 - 1)
    def _():
      fire_group(goff + 48, 1, sem_q1)

    pltpu.make_async_copy(
        id_out.at[phase], oid_hbm.at[pl.ds(base + goff, 32)], sem_w).start()
    return c
  lax.fori_loop(0, NQ // 32, pair_body, 0)

  for _ in range(2):
    pltpu.make_async_copy(
        id_out.at[0], oid_hbm.at[pl.ds(base, 32)], sem_w).wait()


@jax.jit
def _run(query_ids, id_table, cat_table, brand_table, tags_table,
         buf_category, buf_brand, buf_tags):
  mesh = plsc.VectorSubcoreMesh(core_axis_name="c", subcore_axis_name="s")

  cat128 = jnp.pad(cat_table, ((0, 0), (0, 128 - FEAT_DIM)))
  brand128 = jnp.pad(brand_table, ((0, 0), (0, 128 - FEAT_DIM)))

  ocat, obrand, otags = pl.kernel(
      _kernel_a,
      out_type=(
          jax.ShapeDtypeStruct((B, 128), jnp.float32),
          jax.ShapeDtypeStruct((B, 128), jnp.float32),
          jax.ShapeDtypeStruct((B, 128), jnp.float32),
      ),
      mesh=mesh,
      compiler_params=pltpu.CompilerParams(use_tc_tiling_on_sc=False),
      scratch_types=[
          pltpu.VMEM((NQ,), jnp.int32),
          pltpu.VMEM((NQ,), jnp.int32),
          pltpu.VMEM((NQ,), jnp.int32),
          pltpu.VMEM((NQ * MAX_LEN,), jnp.int32),
          pltpu.VMEM((NQ * MAX_LEN,), jnp.int32),
          pltpu.VMEM((2, IC, 128), jnp.float32),
          pltpu.VMEM((2, IC, 128), jnp.float32),
          pltpu.VMEM((NQ, FEAT_DIM), jnp.float32),
          pltpu.VMEM((2, TQ * MAX_LEN, FEAT_DIM), jnp.float32),
          pltpu.SemaphoreType.DMA,
          pltpu.SemaphoreType.DMA,
          pltpu.SemaphoreType.DMA,
          pltpu.SemaphoreType.DMA,
      ],
  )(query_ids, buf_tags.T.reshape(-1), buf_category, buf_brand,
    cat128, brand128, tags_table)

  oid = pl.kernel(
      _kernel_b,
      out_type=jax.ShapeDtypeStruct((B, ID_DIM), jnp.float32),
      mesh=mesh,
      scratch_types=[
          pltpu.VMEM((NQ,), jnp.int32),
          pltpu.VMEM((2, 32, ID_DIM), jnp.float32),
          pltpu.VMEM((32, 8, ID_DIM), jnp.float32),
          pltpu.SemaphoreType.DMA,
          pltpu.SemaphoreType.DMA,
          pltpu.SemaphoreType.DMA,
      ],
  )(query_ids, id_table)

  return jnp.concatenate(
      [oid, ocat[:, :FEAT_DIM], obrand[:, :FEAT_DIM],
       otags[:, :FEAT_DIM]], axis=-1)


def kernel(query_ids, id_table, cat_table, brand_table, tags_table,
           buf_category, buf_brand, buf_tags):
  return _run(query_ids.astype(jnp.int32), id_table, cat_table, brand_table,
              tags_table, buf_category.astype(jnp.int32),
              buf_brand.astype(jnp.int32), buf_tags.astype(jnp.int32))


# speedup vs baseline: 3.2126x; 1.0002x over previous
import jax
import jax.numpy as jnp
from jax import lax
from jax.experimental import pallas as pl
from jax.experimental.pallas import tpu as pltpu
from jax.experimental.pallas import tpu_sc as plsc

B = 16384
NE = 1000000
ID_DIM = 64
FEAT_DIM = 32
MAX_LEN = 20
NC = 2
NS = 16
NW = NC * NS
NQ = B // NW
IC = 128
TQ = 16
ICT = 80
NTC = NQ // TQ


def _kernel_a(qid_hbm, btflat_hbm, bufcat_hbm, bufbrand_hbm,
              cattab_hbm, brandtab_hbm, tagstab_hbm,
              ocat_hbm, obrand_hbm, otags_hbm,
              qid_v, cat_idx, brand_idx, flat_pos, tif_v,
              cat_rows, brand_rows, tags_acc, tchunk,
              sem_i, sem, sem_t, sem_w):
  wid = lax.axis_index("s") * NC + lax.axis_index("c")
  base = wid * NQ

  pltpu.sync_copy(qid_hbm.at[pl.ds(base, NQ)], qid_v)

  def fire_ent(j, c):
    pltpu.make_async_copy(
        bufcat_hbm.at[qid_v.at[pl.ds(j * IC, IC)]],
        cat_idx.at[pl.ds(j * IC, IC)], sem_i).start()
    pltpu.make_async_copy(
        bufbrand_hbm.at[qid_v.at[pl.ds(j * IC, IC)]],
        brand_idx.at[pl.ds(j * IC, IC)], sem_i).start()
    return c
  lax.fori_loop(0, NQ // IC, fire_ent, 0)

  lane16 = lax.iota(jnp.int32, 16)

  def fp_body(g, c):
    qv = qid_v[pl.ds(g * 16, 16)]
    pos0 = (g * 16 + lane16) * MAX_LEN
    for s in range(MAX_LEN):
      plsc.store_scatter(flat_pos, [pos0 + s], qv + s * NE)
    return c
  lax.fori_loop(0, NQ // 16, fp_body, 0)

  def fire_tif(k, c):
    pltpu.make_async_copy(
        btflat_hbm.at[flat_pos.at[pl.ds(k * IC, IC)]],
        tif_v.at[pl.ds(k * IC, IC)], sem_i).start()
    return c
  lax.fori_loop(0, (NQ * MAX_LEN) // IC, fire_tif, 0)

  def wait_ent(j, c):
    pltpu.make_async_copy(
        bufcat_hbm.at[qid_v.at[pl.ds(0, IC)]],
        cat_idx.at[pl.ds(0, IC)], sem_i).wait()
    return c
  lax.fori_loop(0, 2 * (NQ // IC), wait_ent, 0)

  def fire_feat(j, buf):
    pltpu.make_async_copy(
        cattab_hbm.at[cat_idx.at[pl.ds(j * IC, IC)]],
        cat_rows.at[buf], sem).start()
    pltpu.make_async_copy(
        brandtab_hbm.at[brand_idx.at[pl.ds(j * IC, IC)]],
        brand_rows.at[buf], sem).start()

  fire_feat(0, 0)

  def feat_body(j, c):
    buf = lax.bitwise_and(j, 1)

    @pl.when(j >= 2)
    def _():
      pltpu.make_async_copy(
          cat_rows.at[0], ocat_hbm.at[pl.ds(base, IC)], sem_w).wait()
      pltpu.make_async_copy(
          brand_rows.at[0], obrand_hbm.at[pl.ds(base, IC)], sem_w).wait()

    @pl.when(j < NQ // IC - 1)
    def _():
      fire_feat(j + 1, 1 - buf)

    pltpu.make_async_copy(
        cattab_hbm.at[cat_idx.at[pl.ds(0, IC)]],
        cat_rows.at[0], sem).wait()
    pltpu.make_async_copy(
        brandtab_hbm.at[brand_idx.at[pl.ds(0, IC)]],
        brand_rows.at[0], sem).wait()
    pltpu.make_async_copy(
        cat_rows.at[buf], ocat_hbm.at[pl.ds(base + j * IC, IC)],
        sem_w).start()
    pltpu.make_async_copy(
        brand_rows.at[buf], obrand_hbm.at[pl.ds(base + j * IC, IC)],
        sem_w).start()
    return c
  lax.fori_loop(0, NQ // IC, feat_body, 0)

  def wait_tif(k, c):
    pltpu.make_async_copy(
        btflat_hbm.at[flat_pos.at[pl.ds(0, IC)]],
        tif_v.at[pl.ds(0, IC)], sem_i).wait()
    return c
  lax.fori_loop(0, (NQ * MAX_LEN) // IC, wait_tif, 0)

  NB = (TQ * MAX_LEN) // ICT

  def fire_chunk(c, buf):
    def fire_k(k, cc):
      pltpu.make_async_copy(
          tagstab_hbm.at[tif_v.at[pl.ds(c * TQ * MAX_LEN + k * ICT, ICT)]],
          tchunk.at[buf, pl.ds(k * ICT, ICT)], sem_t).start()
      return cc
    lax.fori_loop(0, NB, fire_k, 0)

  def wait_chunk():
    def wait_k(k, cc):
      pltpu.make_async_copy(
          tagstab_hbm.at[tif_v.at[pl.ds(0, ICT)]],
          tchunk.at[0, pl.ds(0, ICT)], sem_t).wait()
      return cc
    lax.fori_loop(0, NB, wait_k, 0)

  fire_chunk(0, 0)

  def chunk_body(c, carry):
    buf = lax.bitwise_and(c, 1)

    @pl.when(c < NTC - 1)
    def _():
      fire_chunk(c + 1, 1 - buf)
    wait_chunk()

    def red(q, cc):
      a0 = tchunk[buf, q * MAX_LEN, pl.ds(0, 16)]
      a1 = tchunk[buf, q * MAX_LEN, pl.ds(16, 16)]
      for s in range(1, MAX_LEN):
        a0 = a0 + tchunk[buf, q * MAX_LEN + s, pl.ds(0, 16)]
        a1 = a1 + tchunk[buf, q * MAX_LEN + s, pl.ds(16, 16)]
      tags_acc[c * TQ + q, pl.ds(0, 16)] = a0
      tags_acc[c * TQ + q, pl.ds(16, 16)] = a1
      return cc
    lax.fori_loop(0, TQ, red, 0)
    return carry
  lax.fori_loop(0, NTC, chunk_body, 0)

  for _ in range(2):
    pltpu.make_async_copy(
        cat_rows.at[0], ocat_hbm.at[pl.ds(base, IC)], sem_w).wait()
    pltpu.make_async_copy(
        brand_rows.at[0], obrand_hbm.at[pl.ds(base, IC)], sem_w).wait()
  pltpu.sync_copy(
      tags_acc, otags_hbm.at[pl.ds(base, NQ), pl.ds(0, FEAT_DIM)])


def _kernel_b(qid_hbm, idtab_hbm, oid_hbm,
              qid_v, id_out, id8, sem_q0, sem_q1, sem_w):
  wid = lax.axis_index("s") * NC + lax.axis_index("c")
  base = wid * NQ

  pltpu.sync_copy(qid_hbm.at[pl.ds(base, NQ)], qid_v)

  def fire_group(goff, par, sem_q):
    qv = qid_v[pl.ds(goff, 16)]
    for j in range(16):
      r = qv[j]
      rb = pl.multiple_of(r - lax.bitwise_and(r, 7), 8)
      pltpu.make_async_copy(
          idtab_hbm.at[pl.ds(rb, 8)], id8.at[par * 16 + j], sem_q).start()

  def drain_extract(goff, par, phase, sem_q):
    for j in range(16):
      pltpu.make_async_copy(
          idtab_hbm.at[pl.ds(0, 8)], id8.at[par * 16 + j], sem_q).wait()
    qv = qid_v[pl.ds(goff, 16)]
    for j in range(16):
      slot = par * 16 + j
      sub = lax.bitwise_and(qv[j], 7)
      for k in range(ID_DIM // 16):
        id_out[phase, par * 16 + j, pl.ds(k * 16, 16)] = (
            id8[slot, sub, pl.ds(k * 16, 16)])

  fire_group(0, 0, sem_q0)
  fire_group(16, 1, sem_q1)

  def pair_body(gg, c):
    goff = gg * 32
    phase = lax.bitwise_and(gg, 1)

    @pl.when(gg >= 2)
    def _():
      pltpu.make_async_copy(
          id_out.at[0], oid_hbm.at[pl.ds(base, 32)], sem_w).wait()

    drain_extract(goff, 0, phase, sem_q0)

    @pl.when(gg < NQ // 32 - 1)
    def _():
      fire_group(goff + 32, 0, sem_q0)
    drain_extract(goff + 16, 1, phase, sem_q1)

    @pl.when(gg < NQ // 32 - 1)
    def _():
      fire_group(goff + 48, 1, sem_q1)

    pltpu.make_async_copy(
        id_out.at[phase], oid_hbm.at[pl.ds(base + goff, 32)], sem_w).start()
    return c
  lax.fori_loop(0, NQ // 32, pair_body, 0)

  for _ in range(2):
    pltpu.make_async_copy(
        id_out.at[0], oid_hbm.at[pl.ds(base, 32)], sem_w).wait()


@jax.jit
def _run(query_ids, id_table, cat_table, brand_table, tags_table,
         buf_category, buf_brand, buf_tags):
  mesh = plsc.VectorSubcoreMesh(core_axis_name="c", subcore_axis_name="s")

  cat128 = jnp.pad(cat_table, ((0, 0), (0, 128 - FEAT_DIM)))
  brand128 = jnp.pad(brand_table, ((0, 0), (0, 128 - FEAT_DIM)))

  ocat, obrand, otags = pl.kernel(
      _kernel_a,
      out_type=(
          jax.ShapeDtypeStruct((B, 128), jnp.float32),
          jax.ShapeDtypeStruct((B, 128), jnp.float32),
          jax.ShapeDtypeStruct((B, 128), jnp.float32),
      ),
      mesh=mesh,
      compiler_params=pltpu.CompilerParams(
          use_tc_tiling_on_sc=False, needs_layout_passes=False),
      scratch_types=[
          pltpu.VMEM((NQ,), jnp.int32),
          pltpu.VMEM((NQ,), jnp.int32),
          pltpu.VMEM((NQ,), jnp.int32),
          pltpu.VMEM((NQ * MAX_LEN,), jnp.int32),
          pltpu.VMEM((NQ * MAX_LEN,), jnp.int32),
          pltpu.VMEM((2, IC, 128), jnp.float32),
          pltpu.VMEM((2, IC, 128), jnp.float32),
          pltpu.VMEM((NQ, FEAT_DIM), jnp.float32),
          pltpu.VMEM((2, TQ * MAX_LEN, FEAT_DIM), jnp.float32),
          pltpu.SemaphoreType.DMA,
          pltpu.SemaphoreType.DMA,
          pltpu.SemaphoreType.DMA,
          pltpu.SemaphoreType.DMA,
      ],
  )(query_ids, buf_tags.T.reshape(-1), buf_category, buf_brand,
    cat128, brand128, tags_table)

  oid = pl.kernel(
      _kernel_b,
      out_type=jax.ShapeDtypeStruct((B, ID_DIM), jnp.float32),
      mesh=mesh,
      scratch_types=[
          pltpu.VMEM((NQ,), jnp.int32),
          pltpu.VMEM((2, 32, ID_DIM), jnp.float32),
          pltpu.VMEM((32, 8, ID_DIM), jnp.float32),
          pltpu.SemaphoreType.DMA,
          pltpu.SemaphoreType.DMA,
          pltpu.SemaphoreType.DMA,
      ],
  )(query_ids, id_table)

  return jnp.concatenate(
      [oid, ocat[:, :FEAT_DIM], obrand[:, :FEAT_DIM],
       otags[:, :FEAT_DIM]], axis=-1)


def kernel(query_ids, id_table, cat_table, brand_table, tags_table,
           buf_category, buf_brand, buf_tags):
  return _run(query_ids.astype(jnp.int32), id_table, cat_table, brand_table,
              tags_table, buf_category.astype(jnp.int32),
              buf_brand.astype(jnp.int32), buf_tags.astype(jnp.int32))


# speedup vs baseline: 7.9788x; 2.4836x over previous
import jax
import jax.numpy as jnp
from jax import lax
from jax.experimental import pallas as pl
from jax.experimental.pallas import tpu as pltpu
from jax.experimental.pallas import tpu_sc as plsc

B = 16384
ID_DIM = 64
FEAT_DIM = 32
MAX_LEN = 20
NC = 2
NS = 16
NW = NC * NS
NQ = B // NW
IC = 128
RING = 8
TQ = 16
NTC = NQ // TQ


def _stage1(qid_hbm, idtab_hbm, buftags_hbm, bufcat_hbm, bufbrand_hbm,
            oid_hbm, otif_hbm, ocat_hbm, obrand_hbm,
            qid_v, cat_idx, brand_idx, id_out, tags_if,
            id8, tb8, sem, sem_q0, sem_q1, sem_w):
  wid = lax.axis_index("s") * NC + lax.axis_index("c")
  base = wid * NQ

  pltpu.sync_copy(qid_hbm.at[pl.ds(base, NQ)], qid_v)

  def fire_ent(j, c):
    pltpu.make_async_copy(
        bufcat_hbm.at[qid_v.at[pl.ds(j * IC, IC)]],
        cat_idx.at[pl.ds(j * IC, IC)], sem).start()
    pltpu.make_async_copy(
        bufbrand_hbm.at[qid_v.at[pl.ds(j * IC, IC)]],
        brand_idx.at[pl.ds(j * IC, IC)], sem).start()
    return c
  lax.fori_loop(0, NQ // IC, fire_ent, 0)

  def fire_group(goff, par, sem_q):
    qv = qid_v[pl.ds(goff, 16)]
    for j in range(16):
      r = qv[j]
      rb = pl.multiple_of(r - lax.bitwise_and(r, 7), 8)
      pltpu.make_async_copy(
          idtab_hbm.at[pl.ds(rb, 8)], id8.at[par * 16 + j], sem_q).start()
      pltpu.make_async_copy(
          buftags_hbm.at[pl.ds(rb, 8)], tb8.at[par * 16 + j], sem_q).start()

  def drain_extract(goff, par, phase, sem_q):
    for j in range(16):
      pltpu.make_async_copy(
          idtab_hbm.at[pl.ds(0, 8)], id8.at[par * 16 + j], sem_q).wait()
      pltpu.make_async_copy(
          buftags_hbm.at[pl.ds(0, 8)], tb8.at[par * 16 + j], sem_q).wait()
    qv = qid_v[pl.ds(goff, 16)]
    for j in range(16):
      slot = par * 16 + j
      sub = lax.bitwise_and(qv[j], 7)
      for k in range(ID_DIM // 16):
        id_out[phase, par * 16 + j, pl.ds(k * 16, 16)] = (
            id8[slot, sub, pl.ds(k * 16, 16)])
      tags_if[pl.ds((goff + j) * MAX_LEN, 16)] = tb8[slot, sub, pl.ds(0, 16)]
      tags_if[pl.ds((goff + j) * MAX_LEN + 4, 16)] = tb8[slot, sub,
                                                        pl.ds(4, 16)]

  fire_group(0, 0, sem_q0)
  fire_group(16, 1, sem_q1)

  def pair_body(gg, c):
    goff = gg * 32
    phase = lax.bitwise_and(gg, 1)

    @pl.when(gg >= 2)
    def _():
      pltpu.make_async_copy(
          id_out.at[0], oid_hbm.at[pl.ds(base, 32)], sem_w).wait()

    drain_extract(goff, 0, phase, sem_q0)

    @pl.when(gg < NQ // 32 - 1)
    def _():
      fire_group(goff + 32, 0, sem_q0)
    drain_extract(goff + 16, 1, phase, sem_q1)

    @pl.when(gg < NQ // 32 - 1)
    def _():
      fire_group(goff + 48, 1, sem_q1)

    pltpu.make_async_copy(
        id_out.at[phase], oid_hbm.at[pl.ds(base + goff, 32)], sem_w).start()
    return c
  lax.fori_loop(0, NQ // 32, pair_body, 0)

  for _ in range(2):
    pltpu.make_async_copy(
        id_out.at[0], oid_hbm.at[pl.ds(base, 32)], sem_w).wait()

  def wait_ent(j, c):
    pltpu.make_async_copy(
        bufcat_hbm.at[qid_v.at[pl.ds(0, IC)]],
        cat_idx.at[pl.ds(0, IC)], sem).wait()
    pltpu.make_async_copy(
        bufbrand_hbm.at[qid_v.at[pl.ds(0, IC)]],
        brand_idx.at[pl.ds(0, IC)], sem).wait()
    return c
  lax.fori_loop(0, NQ // IC, wait_ent, 0)

  w1 = pltpu.make_async_copy(
      tags_if, otif_hbm.at[pl.ds(base * MAX_LEN, NQ * MAX_LEN)], sem_w)
  w2 = pltpu.make_async_copy(cat_idx, ocat_hbm.at[pl.ds(base, NQ)], sem_w)
  w3 = pltpu.make_async_copy(brand_idx, obrand_hbm.at[pl.ds(base, NQ)], sem_w)
  w1.start(), w2.start(), w3.start()
  w1.wait(), w2.wait(), w3.wait()


def _stage2(catidx_hbm, brandidx_hbm, tif_hbm,
            cattab_hbm, brandtab_hbm, tagstab_hbm,
            ocat_hbm, obrand_hbm, otags_hbm,
            cat_idx, brand_idx, tif_v, cat_rows, brand_rows, tags_acc,
            tchunk, sem, sem_t, sem_w):
  wid = lax.axis_index("s") * NC + lax.axis_index("c")
  base = wid * NQ

  pltpu.sync_copy(catidx_hbm.at[pl.ds(base, NQ)], cat_idx)
  pltpu.sync_copy(brandidx_hbm.at[pl.ds(base, NQ)], brand_idx)
  pltpu.sync_copy(
      tif_hbm.at[pl.ds(base * MAX_LEN, NQ * MAX_LEN)], tif_v)

  def fire_feat(j, buf):
    pltpu.make_async_copy(
        cattab_hbm.at[cat_idx.at[pl.ds(j * IC, IC)]],
        cat_rows.at[buf], sem).start()
    pltpu.make_async_copy(
        brandtab_hbm.at[brand_idx.at[pl.ds(j * IC, IC)]],
        brand_rows.at[buf], sem).start()

  fire_feat(0, 0)

  def feat_body(j, c):
    buf = lax.bitwise_and(j, 1)

    @pl.when(j >= 2)
    def _():
      pltpu.make_async_copy(
          cat_rows.at[0], ocat_hbm.at[pl.ds(base, IC)], sem_w).wait()
      pltpu.make_async_copy(
          brand_rows.at[0], obrand_hbm.at[pl.ds(base, IC)], sem_w).wait()

    @pl.when(j < NQ // IC - 1)
    def _():
      fire_feat(j + 1, 1 - buf)

    pltpu.make_async_copy(
        cattab_hbm.at[cat_idx.at[pl.ds(0, IC)]],
        cat_rows.at[0], sem).wait()
    pltpu.make_async_copy(
        brandtab_hbm.at[brand_idx.at[pl.ds(0, IC)]],
        brand_rows.at[0], sem).wait()
    pltpu.make_async_copy(
        cat_rows.at[buf], ocat_hbm.at[pl.ds(base + j * IC, IC)],
        sem_w).start()
    pltpu.make_async_copy(
        brand_rows.at[buf], obrand_hbm.at[pl.ds(base + j * IC, IC)],
        sem_w).start()
    return c
  lax.fori_loop(0, NQ // IC, feat_body, 0)

  ICT = 80
  NB = (TQ * MAX_LEN) // ICT

  def fire_chunk(c, buf):
    def fire_k(k, cc):
      pltpu.make_async_copy(
          tagstab_hbm.at[tif_v.at[pl.ds(c * TQ * MAX_LEN + k * ICT, ICT)]],
          tchunk.at[buf, pl.ds(k * ICT, ICT)], sem_t).start()
      return cc
    lax.fori_loop(0, NB, fire_k, 0)

  def wait_chunk():
    def wait_k(k, cc):
      pltpu.make_async_copy(
          tagstab_hbm.at[tif_v.at[pl.ds(0, ICT)]],
          tchunk.at[0, pl.ds(0, ICT)], sem_t).wait()
      return cc
    lax.fori_loop(0, NB, wait_k, 0)

  fire_chunk(0, 0)

  def chunk_body(c, carry):
    buf = lax.bitwise_and(c, 1)

    @pl.when(c < NTC - 1)
    def _():
      fire_chunk(c + 1, 1 - buf)
    wait_chunk()

    def red(q, cc):
      a0 = tchunk[buf, q * MAX_LEN, pl.ds(0, 16)]
      a1 = tchunk[buf, q * MAX_LEN, pl.ds(16, 16)]
      for s in range(1, MAX_LEN):
        a0 = a0 + tchunk[buf, q * MAX_LEN + s, pl.ds(0, 16)]
        a1 = a1 + tchunk[buf, q * MAX_LEN + s, pl.ds(16, 16)]
      tags_acc[c * TQ + q, pl.ds(0, 16)] = a0
      tags_acc[c * TQ + q, pl.ds(16, 16)] = a1
      return cc
    lax.fori_loop(0, TQ, red, 0)
    return carry
  lax.fori_loop(0, NTC, chunk_body, 0)

  for _ in range(2):
    pltpu.make_async_copy(
        cat_rows.at[0], ocat_hbm.at[pl.ds(base, IC)], sem_w).wait()
    pltpu.make_async_copy(
        brand_rows.at[0], obrand_hbm.at[pl.ds(base, IC)], sem_w).wait()
  pltpu.sync_copy(
      tags_acc, otags_hbm.at[pl.ds(base, NQ), pl.ds(0, FEAT_DIM)])


@jax.jit
def _run(query_ids, id_table, cat_table, brand_table, tags_table,
         buf_category, buf_brand, buf_tags):
  mesh = plsc.VectorSubcoreMesh(core_axis_name="c", subcore_axis_name="s")
  id_emb, tags_if, cat_idx, brand_idx = pl.kernel(
      _stage1,
      out_type=(
          jax.ShapeDtypeStruct((B, ID_DIM), jnp.float32),
          jax.ShapeDtypeStruct((B * MAX_LEN,), jnp.int32),
          jax.ShapeDtypeStruct((B,), jnp.int32),
          jax.ShapeDtypeStruct((B,), jnp.int32),
      ),
      mesh=mesh,
      scratch_types=[
          pltpu.VMEM((NQ,), jnp.int32),
          pltpu.VMEM((NQ,), jnp.int32),
          pltpu.VMEM((NQ,), jnp.int32),
          pltpu.VMEM((2, 32, ID_DIM), jnp.float32),
          pltpu.VMEM((NQ * MAX_LEN,), jnp.int32),
          pltpu.VMEM((32, 8, ID_DIM), jnp.float32),
          pltpu.VMEM((32, 8, MAX_LEN), jnp.int32),
          pltpu.SemaphoreType.DMA,
          pltpu.SemaphoreType.DMA,
          pltpu.SemaphoreType.DMA,
          pltpu.SemaphoreType.DMA,
      ],
  )(query_ids, id_table, buf_tags, buf_category, buf_brand)

  cat128 = jnp.pad(cat_table, ((0, 0), (0, 128 - FEAT_DIM)))
  brand128 = jnp.pad(brand_table, ((0, 0), (0, 128 - FEAT_DIM)))

  ocat, obrand, otags = pl.kernel(
      _stage2,
      out_type=(
          jax.ShapeDtypeStruct((B, 128), jnp.float32),
          jax.ShapeDtypeStruct((B, 128), jnp.float32),
          jax.ShapeDtypeStruct((B, 128), jnp.float32),
      ),
      mesh=mesh,
      compiler_params=pltpu.CompilerParams(use_tc_tiling_on_sc=False),
      scratch_types=[
          pltpu.VMEM((NQ,), jnp.int32),
          pltpu.VMEM((NQ,), jnp.int32),
          pltpu.VMEM((NQ * MAX_LEN,), jnp.int32),
          pltpu.VMEM((2, IC, 128), jnp.float32),
          pltpu.VMEM((2, IC, 128), jnp.float32),
          pltpu.VMEM((NQ, FEAT_DIM), jnp.float32),
          pltpu.VMEM((2, TQ * MAX_LEN, FEAT_DIM), jnp.float32),
          pltpu.SemaphoreType.DMA,
          pltpu.SemaphoreType.DMA,
          pltpu.SemaphoreType.DMA,
      ],
  )(cat_idx, brand_idx, tags_if, cat128, brand128, tags_table)

  return jnp.concatenate(
      [id_emb, ocat[:, :FEAT_DIM], obrand[:, :FEAT_DIM],
       otags[:, :FEAT_DIM]], axis=-1)


def kernel(query_ids, id_table, cat_table, brand_table, tags_table,
           buf_category, buf_brand, buf_tags):
  return _run(query_ids.astype(jnp.int32), id_table, cat_table, brand_table,
              tags_table, buf_category.astype(jnp.int32),
              buf_brand.astype(jnp.int32), buf_tags.astype(jnp.int32))


# speedup vs baseline: 8.0089x; 1.0038x over previous
import jax
import jax.numpy as jnp
from jax import lax
from jax.experimental import pallas as pl
from jax.experimental.pallas import tpu as pltpu
from jax.experimental.pallas import tpu_sc as plsc

B = 16384
ID_DIM = 64
FEAT_DIM = 32
MAX_LEN = 20
NC = 2
NS = 16
NW = NC * NS
NQ = B // NW
IC = 128
RING = 8
TQ = 16
NTC = NQ // TQ


def _stage1a(qid_hbm, buftags_hbm, bufcat_hbm, bufbrand_hbm,
             otif_hbm, ocat_hbm, obrand_hbm,
             qid_v, cat_idx, brand_idx, tags_if,
             tb8, sem, sem_q0, sem_q1, sem_w):
  wid = lax.axis_index("s") * NC + lax.axis_index("c")
  base = wid * NQ

  pltpu.sync_copy(qid_hbm.at[pl.ds(base, NQ)], qid_v)

  def fire_ent(j, c):
    pltpu.make_async_copy(
        bufcat_hbm.at[qid_v.at[pl.ds(j * IC, IC)]],
        cat_idx.at[pl.ds(j * IC, IC)], sem).start()
    pltpu.make_async_copy(
        bufbrand_hbm.at[qid_v.at[pl.ds(j * IC, IC)]],
        brand_idx.at[pl.ds(j * IC, IC)], sem).start()
    return c
  lax.fori_loop(0, NQ // IC, fire_ent, 0)

  def fire_group(goff, par, sem_q):
    qv = qid_v[pl.ds(goff, 16)]
    for j in range(16):
      r = qv[j]
      rb = pl.multiple_of(r - lax.bitwise_and(r, 7), 8)
      pltpu.make_async_copy(
          buftags_hbm.at[pl.ds(rb, 8)], tb8.at[par * 16 + j], sem_q).start()

  def drain_extract(goff, par, sem_q):
    for j in range(16):
      pltpu.make_async_copy(
          buftags_hbm.at[pl.ds(0, 8)], tb8.at[par * 16 + j], sem_q).wait()
    qv = qid_v[pl.ds(goff, 16)]
    for j in range(16):
      slot = par * 16 + j
      sub = lax.bitwise_and(qv[j], 7)
      tags_if[pl.ds((goff + j) * MAX_LEN, 16)] = tb8[slot, sub, pl.ds(0, 16)]
      tags_if[pl.ds((goff + j) * MAX_LEN + 4, 16)] = tb8[slot, sub,
                                                        pl.ds(4, 16)]

  fire_group(0, 0, sem_q0)
  fire_group(16, 1, sem_q1)

  def pair_body(gg, c):
    goff = gg * 32
    drain_extract(goff, 0, sem_q0)

    @pl.when(gg < NQ // 32 - 1)
    def _():
      fire_group(goff + 32, 0, sem_q0)
    drain_extract(goff + 16, 1, sem_q1)

    @pl.when(gg < NQ // 32 - 1)
    def _():
      fire_group(goff + 48, 1, sem_q1)
    return c
  lax.fori_loop(0, NQ // 32, pair_body, 0)

  def wait_ent(j, c):
    pltpu.make_async_copy(
        bufcat_hbm.at[qid_v.at[pl.ds(0, IC)]],
        cat_idx.at[pl.ds(0, IC)], sem).wait()
    pltpu.make_async_copy(
        bufbrand_hbm.at[qid_v.at[pl.ds(0, IC)]],
        brand_idx.at[pl.ds(0, IC)], sem).wait()
    return c
  lax.fori_loop(0, NQ // IC, wait_ent, 0)

  w1 = pltpu.make_async_copy(
      tags_if, otif_hbm.at[pl.ds(base * MAX_LEN, NQ * MAX_LEN)], sem_w)
  w2 = pltpu.make_async_copy(cat_idx, ocat_hbm.at[pl.ds(base, NQ)], sem_w)
  w3 = pltpu.make_async_copy(brand_idx, obrand_hbm.at[pl.ds(base, NQ)], sem_w)
  w1.start(), w2.start(), w3.start()
  w1.wait(), w2.wait(), w3.wait()


def _stage1b(qid_hbm, idtab_hbm, oid_hbm,
             qid_v, id_out, id8, sem_q0, sem_q1, sem_w):
  wid = lax.axis_index("s") * NC + lax.axis_index("c")
  base = wid * NQ

  pltpu.sync_copy(qid_hbm.at[pl.ds(base, NQ)], qid_v)

  def fire_group(goff, par, sem_q):
    qv = qid_v[pl.ds(goff, 16)]
    for j in range(16):
      r = qv[j]
      rb = pl.multiple_of(r - lax.bitwise_and(r, 7), 8)
      pltpu.make_async_copy(
          idtab_hbm.at[pl.ds(rb, 8)], id8.at[par * 16 + j], sem_q).start()

  def drain_extract(goff, par, phase, sem_q):
    for j in range(16):
      pltpu.make_async_copy(
          idtab_hbm.at[pl.ds(0, 8)], id8.at[par * 16 + j], sem_q).wait()
    qv = qid_v[pl.ds(goff, 16)]
    for j in range(16):
      slot = par * 16 + j
      sub = lax.bitwise_and(qv[j], 7)
      for k in range(ID_DIM // 16):
        id_out[phase, par * 16 + j, pl.ds(k * 16, 16)] = (
            id8[slot, sub, pl.ds(k * 16, 16)])

  fire_group(0, 0, sem_q0)
  fire_group(16, 1, sem_q1)

  def pair_body(gg, c):
    goff = gg * 32
    phase = lax.bitwise_and(gg, 1)

    @pl.when(gg >= 2)
    def _():
      pltpu.make_async_copy(
          id_out.at[0], oid_hbm.at[pl.ds(base, 32)], sem_w).wait()

    drain_extract(goff, 0, phase, sem_q0)

    @pl.when(gg < NQ // 32 - 1)
    def _():
      fire_group(goff + 32, 0, sem_q0)
    drain_extract(goff + 16, 1, phase, sem_q1)

    @pl.when(gg < NQ // 32 - 1)
    def _():
      fire_group(goff + 48, 1, sem_q1)

    pltpu.make_async_copy(
        id_out.at[phase], oid_hbm.at[pl.ds(base + goff, 32)], sem_w).start()
    return c
  lax.fori_loop(0, NQ // 32, pair_body, 0)

  for _ in range(2):
    pltpu.make_async_copy(
        id_out.at[0], oid_hbm.at[pl.ds(base, 32)], sem_w).wait()


def _stage2(catidx_hbm, brandidx_hbm, tif_hbm,
            cattab_hbm, brandtab_hbm, tagstab_hbm,
            ocat_hbm, obrand_hbm, otags_hbm,
            cat_idx, brand_idx, tif_v, cat_rows, brand_rows, tags_acc,
            tchunk, sem, sem_t, sem_w):
  wid = lax.axis_index("s") * NC + lax.axis_index("c")
  base = wid * NQ

  pltpu.sync_copy(catidx_hbm.at[pl.ds(base, NQ)], cat_idx)
  pltpu.sync_copy(brandidx_hbm.at[pl.ds(base, NQ)], brand_idx)
  pltpu.sync_copy(
      tif_hbm.at[pl.ds(base * MAX_LEN, NQ * MAX_LEN)], tif_v)

  def fire_feat(j, buf):
    pltpu.make_async_copy(
        cattab_hbm.at[cat_idx.at[pl.ds(j * IC, IC)]],
        cat_rows.at[buf], sem).start()
    pltpu.make_async_copy(
        brandtab_hbm.at[brand_idx.at[pl.ds(j * IC, IC)]],
        brand_rows.at[buf], sem).start()

  fire_feat(0, 0)

  def feat_body(j, c):
    buf = lax.bitwise_and(j, 1)

    @pl.when(j >= 2)
    def _():
      pltpu.make_async_copy(
          cat_rows.at[0], ocat_hbm.at[pl.ds(base, IC)], sem_w).wait()
      pltpu.make_async_copy(
          brand_rows.at[0], obrand_hbm.at[pl.ds(base, IC)], sem_w).wait()

    @pl.when(j < NQ // IC - 1)
    def _():
      fire_feat(j + 1, 1 - buf)

    pltpu.make_async_copy(
        cattab_hbm.at[cat_idx.at[pl.ds(0, IC)]],
        cat_rows.at[0], sem).wait()
    pltpu.make_async_copy(
        brandtab_hbm.at[brand_idx.at[pl.ds(0, IC)]],
        brand_rows.at[0], sem).wait()
    pltpu.make_async_copy(
        cat_rows.at[buf], ocat_hbm.at[pl.ds(base + j * IC, IC)],
        sem_w).start()
    pltpu.make_async_copy(
        brand_rows.at[buf], obrand_hbm.at[pl.ds(base + j * IC, IC)],
        sem_w).start()
    return c
  lax.fori_loop(0, NQ // IC, feat_body, 0)

  ICT = 80
  NB = (TQ * MAX_LEN) // ICT

  def fire_chunk(c, buf):
    def fire_k(k, cc):
      pltpu.make_async_copy(
          tagstab_hbm.at[tif_v.at[pl.ds(c * TQ * MAX_LEN + k * ICT, ICT)]],
          tchunk.at[buf, pl.ds(k * ICT, ICT)], sem_t).start()
      return cc
    lax.fori_loop(0, NB, fire_k, 0)

  def wait_chunk():
    def wait_k(k, cc):
      pltpu.make_async_copy(
          tagstab_hbm.at[tif_v.at[pl.ds(0, ICT)]],
          tchunk.at[0, pl.ds(0, ICT)], sem_t).wait()
      return cc
    lax.fori_loop(0, NB, wait_k, 0)

  fire_chunk(0, 0)

  def chunk_body(c, carry):
    buf = lax.bitwise_and(c, 1)

    @pl.when(c < NTC - 1)
    def _():
      fire_chunk(c + 1, 1 - buf)
    wait_chunk()

    def red(q, cc):
      a0 = tchunk[buf, q * MAX_LEN, pl.ds(0, 16)]
      a1 = tchunk[buf, q * MAX_LEN, pl.ds(16, 16)]
      for s in range(1, MAX_LEN):
        a0 = a0 + tchunk[buf, q * MAX_LEN + s, pl.ds(0, 16)]
        a1 = a1 + tchunk[buf, q * MAX_LEN + s, pl.ds(16, 16)]
      tags_acc[c * TQ + q, pl.ds(0, 16)] = a0
      tags_acc[c * TQ + q, pl.ds(16, 16)] = a1
      return cc
    lax.fori_loop(0, TQ, red, 0)
    return carry
  lax.fori_loop(0, NTC, chunk_body, 0)

  for _ in range(2):
    pltpu.make_async_copy(
        cat_rows.at[0], ocat_hbm.at[pl.ds(base, IC)], sem_w).wait()
    pltpu.make_async_copy(
        brand_rows.at[0], obrand_hbm.at[pl.ds(base, IC)], sem_w).wait()
  pltpu.sync_copy(
      tags_acc, otags_hbm.at[pl.ds(base, NQ), pl.ds(0, FEAT_DIM)])


@jax.jit
def _run(query_ids, id_table, cat_table, brand_table, tags_table,
         buf_category, buf_brand, buf_tags):
  mesh = plsc.VectorSubcoreMesh(core_axis_name="c", subcore_axis_name="s")
  tags_if, cat_idx, brand_idx = pl.kernel(
      _stage1a,
      out_type=(
          jax.ShapeDtypeStruct((B * MAX_LEN,), jnp.int32),
          jax.ShapeDtypeStruct((B,), jnp.int32),
          jax.ShapeDtypeStruct((B,), jnp.int32),
      ),
      mesh=mesh,
      scratch_types=[
          pltpu.VMEM((NQ,), jnp.int32),
          pltpu.VMEM((NQ,), jnp.int32),
          pltpu.VMEM((NQ,), jnp.int32),
          pltpu.VMEM((NQ * MAX_LEN,), jnp.int32),
          pltpu.VMEM((32, 8, MAX_LEN), jnp.int32),
          pltpu.SemaphoreType.DMA,
          pltpu.SemaphoreType.DMA,
          pltpu.SemaphoreType.DMA,
          pltpu.SemaphoreType.DMA,
      ],
  )(query_ids, buf_tags, buf_category, buf_brand)

  cat128 = jnp.pad(cat_table, ((0, 0), (0, 128 - FEAT_DIM)))
  brand128 = jnp.pad(brand_table, ((0, 0), (0, 128 - FEAT_DIM)))

  ocat, obrand, otags = pl.kernel(
      _stage2,
      out_type=(
          jax.ShapeDtypeStruct((B, 128), jnp.float32),
          jax.ShapeDtypeStruct((B, 128), jnp.float32),
          jax.ShapeDtypeStruct((B, 128), jnp.float32),
      ),
      mesh=mesh,
      compiler_params=pltpu.CompilerParams(use_tc_tiling_on_sc=False),
      scratch_types=[
          pltpu.VMEM((NQ,), jnp.int32),
          pltpu.VMEM((NQ,), jnp.int32),
          pltpu.VMEM((NQ * MAX_LEN,), jnp.int32),
          pltpu.VMEM((2, IC, 128), jnp.float32),
          pltpu.VMEM((2, IC, 128), jnp.float32),
          pltpu.VMEM((NQ, FEAT_DIM), jnp.float32),
          pltpu.VMEM((2, TQ * MAX_LEN, FEAT_DIM), jnp.float32),
          pltpu.SemaphoreType.DMA,
          pltpu.SemaphoreType.DMA,
          pltpu.SemaphoreType.DMA,
      ],
  )(cat_idx, brand_idx, tags_if, cat128, brand128, tags_table)

  id_emb = pl.kernel(
      _stage1b,
      out_type=jax.ShapeDtypeStruct((B, ID_DIM), jnp.float32),
      mesh=mesh,
      scratch_types=[
          pltpu.VMEM((NQ,), jnp.int32),
          pltpu.VMEM((2, 32, ID_DIM), jnp.float32),
          pltpu.VMEM((32, 8, ID_DIM), jnp.float32),
          pltpu.SemaphoreType.DMA,
          pltpu.SemaphoreType.DMA,
          pltpu.SemaphoreType.DMA,
      ],
  )(query_ids, id_table)

  return jnp.concatenate(
      [id_emb, ocat[:, :FEAT_DIM], obrand[:, :FEAT_DIM],
       otags[:, :FEAT_DIM]], axis=-1)


def kernel(query_ids, id_table, cat_table, brand_table, tags_table,
           buf_category, buf_brand, buf_tags):
  return _run(query_ids.astype(jnp.int32), id_table, cat_table, brand_table,
              tags_table, buf_category.astype(jnp.int32),
              buf_brand.astype(jnp.int32), buf_tags.astype(jnp.int32))
